# baseline (speedup 1.0000x reference)
import jax
import jax.numpy as jnp
from jax import lax
from jax.experimental import pallas as pl
from jax.experimental.pallas import tpu as pltpu

_B, _N, _D, _K, _E, _HW = 8, 576, 256, 16384, 598, 384
_BN = _B * _N
_BNT = 512
_NBN = _BN // _BNT
_KT = 1024
_NK = _K // _KT
_CARRY = 4096 // _KT


def _vq_kernel(z_ref, cb_ref, cond_ref, z2_ref, zneg_ref, c2_ref, col_ref,
               bv_ref, bi_ref, all_ref):
    i = pl.program_id(0)
    k = pl.program_id(1)

    @pl.when(k == 0)
    def _():
        zb = z_ref[...]
        z2_ref[...] = jnp.sum(zb * zb, axis=1, keepdims=True)
        zneg_ref[...] = (zb * -2.0).astype(jnp.bfloat16)

    cb = cb_ref[pl.ds(pl.multiple_of(k * _KT, _KT), _KT), :]

    @pl.when(i == 0)
    def _():
        c2_ref[k] = jnp.sum(cb * cb, axis=1, keepdims=True).reshape(1, _KT)

    @pl.when((i == 0) & (k == 0))
    def _():
        col_ref[...] = lax.broadcasted_iota(jnp.int32, (_BNT, _KT), 1)

    dots2 = lax.dot_general(zneg_ref[...], cb.astype(jnp.bfloat16),
                            (((1,), (1,)), ((), ())),
                            preferred_element_type=jnp.float32)
    dist = (z2_ref[...] + c2_ref[k]) + dots2

    mv = jnp.min(dist, axis=1, keepdims=True)
    mi = jnp.min(jnp.where(dist == mv, col_ref[...], _K),
                 axis=1, keepdims=True) + k * _KT

    @pl.when(k == 0)
    def _():
        bv_ref[...] = mv
        bi_ref[...] = mi

    @pl.when(k != 0)
    def _():
        upd = mv < bv_ref[...]
        bv_ref[...] = jnp.where(upd, mv, bv_ref[...])
        bi_ref[...] = jnp.where(upd, mi, bi_ref[...])

    @pl.when(k % _CARRY == _CARRY - 1)
    def _():
        bv_ref[...] = bv_ref[...].astype(jnp.bfloat16).astype(jnp.float32)

    @pl.when(k == _NK - 1)
    def _():
        all_ref[i] = bi_ref[...].astype(jnp.float32)

    @pl.when((i == _NBN - 1) & (k == _NK - 1))
    def _():
        fi = all_ref[...].reshape(_BN, 1)
        lo = jnp.min(fi)
        hi = jnp.max(fi)
        cond_ref[...] = (fi - lo) / (hi - lo)


def _emb_kernel(depth_ref, t_ref, w_ref, b_ref, out_ref):
    d = depth_ref[...]
    alpha = jnp.max(d, axis=1, keepdims=True)
    alpha = jnp.where(alpha == 0.0, 1e-4, alpha)
    ts = t_ref[...] * alpha
    emb = lax.dot_general(ts, w_ref[...], (((1,), (0,)), ((), ())),
                          preferred_element_type=jnp.float32) + b_ref[...]
    lo = jnp.min(emb)
    hi = jnp.max(emb)
    out_ref[...] = (emb - lo) / (hi - lo)


def kernel(z, codebook, depth, t, W, b):
    zf = z.reshape(_BN, _D)

    cond = pl.pallas_call(
        _vq_kernel,
        grid=(_NBN, _NK),
        in_specs=[
            pl.BlockSpec((_BNT, _D), lambda i, k: (i, 0)),
            pl.BlockSpec((_K, _D), lambda i, k: (0, 0)),
        ],
        out_specs=pl.BlockSpec((_BN, 1), lambda i, k: (0, 0)),
        out_shape=jax.ShapeDtypeStruct((_BN, 1), jnp.float32),
        scratch_shapes=[
            pltpu.VMEM((_BNT, 1), jnp.float32),
            pltpu.VMEM((_BNT, _D), jnp.bfloat16),
            pltpu.VMEM((_NK, 1, _KT), jnp.float32),
            pltpu.VMEM((_BNT, _KT), jnp.int32),
            pltpu.VMEM((_BNT, 1), jnp.float32),
            pltpu.VMEM((_BNT, 1), jnp.int32),
            pltpu.VMEM((_NBN, _BNT, 1), jnp.float32),
        ],
    )(zf, codebook)

    emb = pl.pallas_call(
        _emb_kernel,
        in_specs=[
            pl.BlockSpec((_B, _HW * _HW), lambda: (0, 0)),
            pl.BlockSpec((_B, 3), lambda: (0, 0)),
            pl.BlockSpec((3, _E), lambda: (0, 0)),
            pl.BlockSpec((1, _E), lambda: (0, 0)),
        ],
        out_specs=pl.BlockSpec((_B, _E), lambda: (0, 0)),
        out_shape=jax.ShapeDtypeStruct((_B, _E), jnp.float32),
    )(depth.reshape(_B, _HW * _HW), t, W, b.reshape(1, _E))

    condition = cond.reshape(_B, _N)[:, None, :]
    embeddings_n = emb[:, None, :]
    return (condition, embeddings_n)


# speedup vs baseline: 1.3146x; 1.3146x over previous
import jax
import jax.numpy as jnp
from jax import lax
from jax.experimental import pallas as pl
from jax.experimental.pallas import tpu as pltpu

_B, _N, _D, _K, _E, _HW = 8, 576, 256, 16384, 598, 384
_BN = _B * _N
_BNT = 1152
_NBN = _BN // _BNT
_KT = 1024
_NK = _K // _KT
_CARRY = 4096 // _KT


def _vq_kernel(z_ref, cb_ref, cond_ref, z2_ref, zneg_ref, bv_ref, bi_ref,
               all_ref):
    i = pl.program_id(0)
    k = pl.program_id(1)

    @pl.when(k == 0)
    def _():
        zb = z_ref[...]
        z2_ref[...] = jnp.sum(zb * zb, axis=1, keepdims=True)
        zneg_ref[...] = (zb * -2.0).astype(jnp.bfloat16)

    cb = cb_ref[pl.ds(pl.multiple_of(k * _KT, _KT), _KT), :]
    dots2 = lax.dot_general(zneg_ref[...], cb.astype(jnp.bfloat16),
                            (((1,), (1,)), ((), ())),
                            preferred_element_type=jnp.float32)
    c2 = jnp.sum(cb * cb, axis=1)
    dist = (z2_ref[...] + c2[None, :]) + dots2

    mv = jnp.min(dist, axis=1, keepdims=True)
    col = lax.broadcasted_iota(jnp.int32, (_BNT, _KT), 1)
    mi = jnp.min(jnp.where(dist == mv, col, _K),
                 axis=1, keepdims=True) + k * _KT

    @pl.when(k == 0)
    def _():
        bv_ref[...] = mv
        bi_ref[...] = mi

    @pl.when(k != 0)
    def _():
        upd = mv < bv_ref[...]
        bv_ref[...] = jnp.where(upd, mv, bv_ref[...])
        bi_ref[...] = jnp.where(upd, mi, bi_ref[...])

    @pl.when(k % _CARRY == _CARRY - 1)
    def _():
        bv_ref[...] = bv_ref[...].astype(jnp.bfloat16).astype(jnp.float32)

    @pl.when(k == _NK - 1)
    def _():
        all_ref[i] = bi_ref[...].astype(jnp.float32)

    @pl.when((i == _NBN - 1) & (k == _NK - 1))
    def _():
        fi = all_ref[...].reshape(_BN, 1)
        lo = jnp.min(fi)
        hi = jnp.max(fi)
        cond_ref[...] = (fi - lo) / (hi - lo)


def _emb_kernel(depth_ref, t_ref, w_ref, b_ref, out_ref):
    d = depth_ref[...]
    alpha = jnp.max(d, axis=1, keepdims=True)
    alpha = jnp.where(alpha == 0.0, 1e-4, alpha)
    ts = t_ref[...] * alpha
    emb = lax.dot_general(ts, w_ref[...], (((1,), (0,)), ((), ())),
                          preferred_element_type=jnp.float32) + b_ref[...]
    lo = jnp.min(emb)
    hi = jnp.max(emb)
    out_ref[...] = (emb - lo) / (hi - lo)


def kernel(z, codebook, depth, t, W, b):
    zf = z.reshape(_BN, _D)

    cond = pl.pallas_call(
        _vq_kernel,
        grid=(_NBN, _NK),
        in_specs=[
            pl.BlockSpec((_BNT, _D), lambda i, k: (i, 0)),
            pl.BlockSpec((_K, _D), lambda i, k: (0, 0)),
        ],
        out_specs=pl.BlockSpec((_BN, 1), lambda i, k: (0, 0)),
        out_shape=jax.ShapeDtypeStruct((_BN, 1), jnp.float32),
        scratch_shapes=[
            pltpu.VMEM((_BNT, 1), jnp.float32),
            pltpu.VMEM((_BNT, _D), jnp.bfloat16),
            pltpu.VMEM((_BNT, 1), jnp.float32),
            pltpu.VMEM((_BNT, 1), jnp.int32),
            pltpu.VMEM((_NBN, _BNT, 1), jnp.float32),
        ],
    )(zf, codebook)

    emb = pl.pallas_call(
        _emb_kernel,
        in_specs=[
            pl.BlockSpec((_B, _HW * _HW), lambda: (0, 0)),
            pl.BlockSpec((_B, 3), lambda: (0, 0)),
            pl.BlockSpec((3, _E), lambda: (0, 0)),
            pl.BlockSpec((1, _E), lambda: (0, 0)),
        ],
        out_specs=pl.BlockSpec((_B, _E), lambda: (0, 0)),
        out_shape=jax.ShapeDtypeStruct((_B, _E), jnp.float32),
    )(depth.reshape(_B, _HW * _HW), t, W, b.reshape(1, _E))

    condition = cond.reshape(_B, _N)[:, None, :]
    embeddings_n = emb[:, None, :]
    return (condition, embeddings_n)


# speedup vs baseline: 1.5369x; 1.1691x over previous
import jax
import jax.numpy as jnp
from jax import lax
from jax.experimental import pallas as pl
from jax.experimental.pallas import tpu as pltpu

_B, _N, _D, _K, _E, _HW = 8, 576, 256, 16384, 598, 384
_BN = _B * _N
_BNT = 1152
_NBN = _BN // _BNT
_KT = 2048
_NK = _K // _KT
_CARRY = 4096 // _KT


def _vq_kernel(z_ref, cb_ref, cond_ref, z2_ref, zneg_ref, bv_ref, bi_ref,
               all_ref):
    i = pl.program_id(0)
    k = pl.program_id(1)

    @pl.when(k == 0)
    def _():
        zb = z_ref[...]
        z2_ref[...] = jnp.sum(zb * zb, axis=1, keepdims=True)
        zneg_ref[...] = (zb * -2.0).astype(jnp.bfloat16)

    cb = cb_ref[pl.ds(pl.multiple_of(k * _KT, _KT), _KT), :]
    dots2 = lax.dot_general(zneg_ref[...], cb.astype(jnp.bfloat16),
                            (((1,), (1,)), ((), ())),
                            preferred_element_type=jnp.float32)
    c2 = jnp.sum(cb * cb, axis=1)
    dist = (z2_ref[...] + c2[None, :]) + dots2

    mv = jnp.min(dist, axis=1, keepdims=True)
    col = lax.broadcasted_iota(jnp.int32, (_BNT, _KT), 1)
    mi = jnp.min(jnp.where(dist == mv, col, _K),
                 axis=1, keepdims=True) + k * _KT

    @pl.when(k == 0)
    def _():
        bv_ref[...] = mv
        bi_ref[...] = mi

    @pl.when(k != 0)
    def _():
        upd = mv < bv_ref[...]
        bv_ref[...] = jnp.where(upd, mv, bv_ref[...])
        bi_ref[...] = jnp.where(upd, mi, bi_ref[...])

    @pl.when(k % _CARRY == _CARRY - 1)
    def _():
        bv_ref[...] = bv_ref[...].astype(jnp.bfloat16).astype(jnp.float32)

    @pl.when(k == _NK - 1)
    def _():
        all_ref[i] = bi_ref[...].astype(jnp.float32)

    @pl.when((i == _NBN - 1) & (k == _NK - 1))
    def _():
        fi = all_ref[...].reshape(_BN, 1)
        lo = jnp.min(fi)
        hi = jnp.max(fi)
        cond_ref[...] = (fi - lo) / (hi - lo)


def _emb_kernel(depth_ref, t_ref, w_ref, b_ref, out_ref):
    d = depth_ref[...]
    alpha = jnp.max(d, axis=1, keepdims=True)
    alpha = jnp.where(alpha == 0.0, 1e-4, alpha)
    ts = t_ref[...] * alpha
    emb = lax.dot_general(ts, w_ref[...], (((1,), (0,)), ((), ())),
                          preferred_element_type=jnp.float32) + b_ref[...]
    lo = jnp.min(emb)
    hi = jnp.max(emb)
    out_ref[...] = (emb - lo) / (hi - lo)


def kernel(z, codebook, depth, t, W, b):
    zf = z.reshape(_BN, _D)

    cond = pl.pallas_call(
        _vq_kernel,
        grid=(_NBN, _NK),
        in_specs=[
            pl.BlockSpec((_BNT, _D), lambda i, k: (i, 0)),
            pl.BlockSpec((_K, _D), lambda i, k: (0, 0)),
        ],
        out_specs=pl.BlockSpec((_BN, 1), lambda i, k: (0, 0)),
        out_shape=jax.ShapeDtypeStruct((_BN, 1), jnp.float32),
        scratch_shapes=[
            pltpu.VMEM((_BNT, 1), jnp.float32),
            pltpu.VMEM((_BNT, _D), jnp.bfloat16),
            pltpu.VMEM((_BNT, 1), jnp.float32),
            pltpu.VMEM((_BNT, 1), jnp.int32),
            pltpu.VMEM((_NBN, _BNT, 1), jnp.float32),
        ],
    )(zf, codebook)

    emb = pl.pallas_call(
        _emb_kernel,
        in_specs=[
            pl.BlockSpec((_B, _HW * _HW), lambda: (0, 0)),
            pl.BlockSpec((_B, 3), lambda: (0, 0)),
            pl.BlockSpec((3, _E), lambda: (0, 0)),
            pl.BlockSpec((1, _E), lambda: (0, 0)),
        ],
        out_specs=pl.BlockSpec((_B, _E), lambda: (0, 0)),
        out_shape=jax.ShapeDtypeStruct((_B, _E), jnp.float32),
    )(depth.reshape(_B, _HW * _HW), t, W, b.reshape(1, _E))

    condition = cond.reshape(_B, _N)[:, None, :]
    embeddings_n = emb[:, None, :]
    return (condition, embeddings_n)
